# baseline (speedup 1.0000x reference)
import jax
import jax.numpy as jnp
from jax import lax
from jax.experimental import pallas as pl
from jax.experimental.pallas import tpu as pltpu
from jax.experimental.pallas import tpu_sc as plsc

N, E, D, DE = 10000, 320000, 128, 4
NC, NS = 2, 16
NT = NC * NS
C = 64
NCHUNK = E // C
NPAD = 10240
ROWS_PER_TILE = NPAD // NS
ZC = 128
RING = 4
DSTN = 8
QUADS = (NCHUNK // NT + RING) // RING


def _sc_body(x_hbm, ei_hbm, ea_hbm, wb_hbm, out0_hbm, out1_hbm,
             src_v, ea_v, rows0, rows1, rows2, rows3, dst_v, wb_v, shared,
             isem0, isem1, isem2, isem3, gsem0, gsem1, gsem2, gsem3, ssem):
    cid = lax.axis_index("c")
    sid = lax.axis_index("s")
    wid = sid * NC + cid
    rows = (rows0, rows1, rows2, rows3)
    isem = (isem0, isem1, isem2, isem3)
    gsem = (gsem0, gsem1, gsem2, gsem3)

    pltpu.sync_copy(wb_hbm, wb_v)
    wb16 = wb_v[...]
    ws = [jnp.full((16,), wb16[a], jnp.float32) for a in range(DE)]
    bg = jnp.full((16,), wb16[DE], jnp.float32)
    one = jnp.ones((16,), jnp.float32)

    def _zrow(r, carry):
        for f in range(D // 16):
            rows0[r, pl.ds(f * 16, 16)] = jnp.zeros((16,), jnp.float32)
            rows1[r, pl.ds(f * 16, 16)] = jnp.zeros((16,), jnp.float32)
        return carry

    lax.fori_loop(0, C, _zrow, 0)
    for j in range(ROWS_PER_TILE // ZC):
        r0 = sid * ROWS_PER_TILE + j * ZC
        pltpu.async_copy(rows0, shared.at[pl.ds(r0, C)], isem0)
        pltpu.async_copy(rows1, shared.at[pl.ds(r0 + C, C)], isem1)
    for j in range(ROWS_PER_TILE // ZC):
        r0 = sid * ROWS_PER_TILE + j * ZC
        pltpu.make_async_copy(rows0, shared.at[pl.ds(r0, C)], isem0).wait()
        pltpu.make_async_copy(rows1, shared.at[pl.ds(r0 + C, C)], isem1).wait()
    plsc.subcore_barrier()

    nch = (NCHUNK - 1 - wid) // NT + 1

    def chunk_base(k):
        return (wid + k * NT) * C

    def fetch_idx(k, s, sync):
        base = chunk_base(k)
        copy = pltpu.sync_copy if sync else (
            lambda s_, d_: pltpu.async_copy(s_, d_, isem[s]))
        copy(ei_hbm.at[pl.ds(base, C)], src_v.at[s])
        copy(ei_hbm.at[pl.ds(E + base, C)], dst_v.at[lax.rem(k, DSTN)])
        for a in range(DE):
            copy(ea_hbm.at[pl.ds(a * E + base, C)], ea_v.at[s, a])

    def wait_idx(k, s):
        base = chunk_base(k)
        pltpu.make_async_copy(ei_hbm.at[pl.ds(base, C)],
                              src_v.at[s], isem[s]).wait()
        pltpu.make_async_copy(ei_hbm.at[pl.ds(E + base, C)],
                              dst_v.at[lax.rem(k, DSTN)], isem[s]).wait()
        for a in range(DE):
            pltpu.make_async_copy(ea_hbm.at[pl.ds(a * E + base, C)],
                                  ea_v.at[s, a], isem[s]).wait()

    def scale_rows(s):
        def _scale(j, c2):
            z = bg
            for a in range(DE):
                z = z + ws[a] * ea_v[s, a, pl.ds(j * 16, 16)]
            g16 = one / (one + jnp.exp(-z))
            for t in range(16):
                e = j * 16 + t
                g = jnp.full((16,), g16[t], jnp.float32)
                for f in range(D // 16):
                    sl = pl.ds(f * 16, 16)
                    rows[s][e, sl] = rows[s][e, sl] * g
            return c2

        lax.fori_loop(0, C // 16, _scale, 0)

    def wait_scatter(s):
        pltpu.make_async_copy(rows[s], shared.at[dst_v.at[0]], ssem).wait()

    fetch_idx(0, 0, sync=True)
    pltpu.async_copy(x_hbm.at[src_v.at[0]], rows0, gsem0)

    @pl.when(1 < nch)
    def _():
        fetch_idx(1, 1, sync=False)

    def sub_body(k, s):
        s1 = (s + 1) % RING
        s2 = (s + 2) % RING

        @pl.when(k + 1 < nch)
        def _():
            wait_idx(k + 1, s1)

            @pl.when(k >= 3)
            def _():
                wait_scatter(s1)

            pltpu.async_copy(x_hbm.at[src_v.at[s1]], rows[s1], gsem[s1])

        pltpu.make_async_copy(x_hbm.at[src_v.at[s]], rows[s], gsem[s]).wait()
        scale_rows(s)
        pltpu.async_copy(rows[s], shared.at[dst_v.at[lax.rem(k, DSTN)]], ssem,
                         add=True)

        @pl.when(k + 2 < nch)
        def _():
            fetch_idx(k + 2, s2, sync=False)

    def _quad(m, carry):
        k = m * RING
        for s in range(RING):
            @pl.when(k + s < nch)
            def _():
                sub_body(k + s, s)

        return carry

    lax.fori_loop(0, QUADS, _quad, 0)
    for s in range(RING):
        wait_scatter(s)
    plsc.subcore_barrier()

    out_hbms = (out0_hbm, out1_hbm)
    for c in range(NC):
        @pl.when(cid == c)
        def _():
            for j in range(ROWS_PER_TILE // ZC):
                r0 = sid * ROWS_PER_TILE + j * ZC
                pltpu.async_copy(shared.at[pl.ds(r0, ZC)],
                                 out_hbms[c].at[pl.ds(r0, ZC)], gsem0)
            for j in range(ROWS_PER_TILE // ZC):
                r0 = sid * ROWS_PER_TILE + j * ZC
                pltpu.make_async_copy(shared.at[pl.ds(r0, ZC)],
                                      out_hbms[c].at[pl.ds(r0, ZC)],
                                      gsem0).wait()


def _sc_aggregate(x, ei_flat, ea_t, wb):
    mesh = plsc.VectorSubcoreMesh(core_axis_name="c", subcore_axis_name="s")
    return pl.kernel(
        _sc_body,
        out_type=(jax.ShapeDtypeStruct((NPAD, D), jnp.float32),
                  jax.ShapeDtypeStruct((NPAD, D), jnp.float32)),
        mesh=mesh,
        scratch_types=[
            pltpu.VMEM((RING, C), jnp.int32),
            pltpu.VMEM((RING, DE, C), jnp.float32),
            pltpu.VMEM((C, D), jnp.float32),
            pltpu.VMEM((C, D), jnp.float32),
            pltpu.VMEM((C, D), jnp.float32),
            pltpu.VMEM((C, D), jnp.float32),
            pltpu.VMEM((DSTN, C), jnp.int32),
            pltpu.VMEM((16,), jnp.float32),
            pltpu.VMEM_SHARED((NPAD, D), jnp.float32),
            pltpu.SemaphoreType.DMA,
            pltpu.SemaphoreType.DMA,
            pltpu.SemaphoreType.DMA,
            pltpu.SemaphoreType.DMA,
            pltpu.SemaphoreType.DMA,
            pltpu.SemaphoreType.DMA,
            pltpu.SemaphoreType.DMA,
            pltpu.SemaphoreType.DMA,
            pltpu.SemaphoreType.DMA,
        ],
    )(x, ei_flat, ea_t, wb)


def _xw_body(x_ref, wn_ref, bn_ref, o_ref):
    o_ref[...] = (jnp.dot(x_ref[...], wn_ref[...],
                          preferred_element_type=jnp.float32) + bn_ref[...])


def _xw(x, Wn, bn):
    BN = 2000
    return pl.pallas_call(
        _xw_body,
        grid=(N // BN,),
        in_specs=[
            pl.BlockSpec((BN, D), lambda i: (i, 0)),
            pl.BlockSpec((D, D), lambda i: (0, 0)),
            pl.BlockSpec((1, D), lambda i: (0, 0)),
        ],
        out_specs=pl.BlockSpec((BN, D), lambda i: (i, 0)),
        out_shape=jax.ShapeDtypeStruct((N, D), jnp.float32),
    )(x, Wn[:D], bn.reshape(1, D))


def _out_body(xw_ref, p0_ref, p1_ref, w2_ref, o_ref):
    agg = p0_ref[...] + p1_ref[...]
    o_ref[...] = xw_ref[...] + jnp.dot(agg, w2_ref[...],
                                       preferred_element_type=jnp.float32)


def _node_update(xw, p0, p1, Wn):
    BN = 2000
    return pl.pallas_call(
        _out_body,
        grid=(N // BN,),
        in_specs=[
            pl.BlockSpec((BN, D), lambda i: (i, 0)),
            pl.BlockSpec((BN, D), lambda i: (i, 0)),
            pl.BlockSpec((BN, D), lambda i: (i, 0)),
            pl.BlockSpec((D, D), lambda i: (0, 0)),
        ],
        out_specs=pl.BlockSpec((BN, D), lambda i: (i, 0)),
        out_shape=jax.ShapeDtypeStruct((N, D), jnp.float32),
    )(xw, p0, p1, Wn[D:])


def kernel(x, edge_index, edge_attr, we, beg, Wn, bn):
    ei_flat = edge_index.reshape(2 * E)
    ea_t = edge_attr.T.reshape(DE * E)
    wb = jnp.concatenate([we.reshape(DE), beg,
                          jnp.zeros((16 - DE - 1,), jnp.float32)])
    xw = _xw(x, Wn, bn)
    p0, p1 = _sc_aggregate(x, ei_flat, ea_t, wb)
    return _node_update(xw, p0, p1, Wn)


# speedup vs baseline: 1.2506x; 1.2506x over previous
import jax
import jax.numpy as jnp
from jax import lax
from jax.experimental import pallas as pl
from jax.experimental.pallas import tpu as pltpu
from jax.experimental.pallas import tpu_sc as plsc

N, E, D, DE = 10000, 320000, 128, 4
NC, NS = 2, 16
NT = NC * NS
C = 128
NCHUNK = E // C
NPAD = 10240
ROWS_PER_TILE = NPAD // NS
PAIRS = (NCHUNK // NT + 2) // 2
DSTN = 4


def _sc_body(x_hbm, ei_hbm, ea_hbm, wb_hbm, out0_hbm, out1_hbm,
             src_a, ea_a, rows_a, src_b, ea_b, rows_b, dst_v,
             wb_v, shared, isem_a, isem_b, gsem_a, gsem_b, ssem):
    cid = lax.axis_index("c")
    sid = lax.axis_index("s")
    wid = sid * NC + cid

    pltpu.sync_copy(wb_hbm, wb_v)
    wb16 = wb_v[...]
    ws = [jnp.full((16,), wb16[a], jnp.float32) for a in range(DE)]
    bg = jnp.full((16,), wb16[DE], jnp.float32)
    one = jnp.ones((16,), jnp.float32)

    def _zrow(r, carry):
        for f in range(D // 16):
            rows_a[r, pl.ds(f * 16, 16)] = jnp.zeros((16,), jnp.float32)
        return carry

    lax.fori_loop(0, C, _zrow, 0)
    for j in range(ROWS_PER_TILE // C):
        pltpu.sync_copy(rows_a, shared.at[pl.ds(sid * ROWS_PER_TILE + j * C, C)])
    plsc.subcore_barrier()

    nch = (NCHUNK - 1 - wid) // NT + 1

    def chunk_base(k):
        return (wid + k * NT) * C

    def fetch_idx(k, src_v, ea_v, sem, sync):
        base = chunk_base(k)
        copy = pltpu.sync_copy if sync else (
            lambda s_, d_: pltpu.async_copy(s_, d_, sem))
        copy(ei_hbm.at[pl.ds(base, C)], src_v.at[0])
        copy(ei_hbm.at[pl.ds(E + base, C)], dst_v.at[lax.rem(k, DSTN)])
        copy(ea_hbm.at[:, pl.ds(base, C)], ea_v)

    def wait_idx(k, src_v, ea_v, sem):
        base = chunk_base(k)
        pltpu.make_async_copy(ei_hbm.at[pl.ds(base, C)], src_v.at[0], sem).wait()
        pltpu.make_async_copy(ei_hbm.at[pl.ds(E + base, C)],
                              dst_v.at[lax.rem(k, DSTN)], sem).wait()
        pltpu.make_async_copy(ea_hbm.at[:, pl.ds(base, C)], ea_v, sem).wait()

    def scale_rows(ea_v, rows_v):
        def _scale(j, c2):
            z = bg
            for a in range(DE):
                z = z + ws[a] * ea_v[a, pl.ds(j * 16, 16)]
            g16 = one / (one + jnp.exp(-z))
            for t in range(16):
                e = j * 16 + t
                g = jnp.full((16,), g16[t], jnp.float32)
                for f in range(D // 16):
                    sl = pl.ds(f * 16, 16)
                    rows_v[e, sl] = rows_v[e, sl] * g
            return c2

        lax.fori_loop(0, C // 16, _scale, 0)

    def wait_scatter(rows_v):
        pltpu.make_async_copy(rows_v, shared.at[dst_v.at[0]], ssem).wait()

    fetch_idx(0, src_a, ea_a, isem_a, sync=True)
    pltpu.async_copy(x_hbm.at[src_a.at[0]], rows_a, gsem_a)

    @pl.when(1 < nch)
    def _():
        fetch_idx(1, src_b, ea_b, isem_b, sync=False)

    def sub_body(k, cur, nxt):
        (src_c, ea_c, rows_c, gsem_c, _isem_c) = cur
        (src_n, ea_n, rows_n, gsem_n, isem_n) = nxt

        @pl.when(k + 1 < nch)
        def _():
            wait_idx(k + 1, src_n, ea_n, isem_n)

            @pl.when(k >= 1)
            def _():
                wait_scatter(rows_n)

            pltpu.async_copy(x_hbm.at[src_n.at[0]], rows_n, gsem_n)

        pltpu.make_async_copy(x_hbm.at[src_c.at[0]], rows_c, gsem_c).wait()
        scale_rows(ea_c, rows_c)
        pltpu.async_copy(rows_c, shared.at[dst_v.at[lax.rem(k, DSTN)]], ssem,
                         add=True)

        @pl.when(k + 2 < nch)
        def _():
            fetch_idx(k + 2, src_c, ea_c, _isem_c, sync=False)

    buf_a = (src_a, ea_a, rows_a, gsem_a, isem_a)
    buf_b = (src_b, ea_b, rows_b, gsem_b, isem_b)

    def _pair(m, carry):
        k = m * 2

        @pl.when(k < nch)
        def _():
            sub_body(k, buf_a, buf_b)

        @pl.when(k + 1 < nch)
        def _():
            sub_body(k + 1, buf_b, buf_a)

        return carry

    lax.fori_loop(0, PAIRS, _pair, 0)
    wait_scatter(rows_a)
    wait_scatter(rows_b)
    plsc.subcore_barrier()

    @pl.when(cid == 0)
    def _():
        for j in range(ROWS_PER_TILE // C):
            r0 = sid * ROWS_PER_TILE + j * C
            pltpu.sync_copy(shared.at[pl.ds(r0, C)], out0_hbm.at[pl.ds(r0, C)])

    @pl.when(cid == 1)
    def _():
        for j in range(ROWS_PER_TILE // C):
            r0 = sid * ROWS_PER_TILE + j * C
            pltpu.sync_copy(shared.at[pl.ds(r0, C)], out1_hbm.at[pl.ds(r0, C)])


def _sc_aggregate(x, ei_flat, ea_t, wb):
    mesh = plsc.VectorSubcoreMesh(core_axis_name="c", subcore_axis_name="s")
    return pl.kernel(
        _sc_body,
        out_type=(jax.ShapeDtypeStruct((NPAD, D), jnp.float32),
                  jax.ShapeDtypeStruct((NPAD, D), jnp.float32)),
        mesh=mesh,
        scratch_types=[
            pltpu.VMEM((1, C), jnp.int32),
            pltpu.VMEM((DE, C), jnp.float32),
            pltpu.VMEM((C, D), jnp.float32),
            pltpu.VMEM((1, C), jnp.int32),
            pltpu.VMEM((DE, C), jnp.float32),
            pltpu.VMEM((C, D), jnp.float32),
            pltpu.VMEM((DSTN, C), jnp.int32),
            pltpu.VMEM((16,), jnp.float32),
            pltpu.VMEM_SHARED((NPAD, D), jnp.float32),
            pltpu.SemaphoreType.DMA,
            pltpu.SemaphoreType.DMA,
            pltpu.SemaphoreType.DMA,
            pltpu.SemaphoreType.DMA,
            pltpu.SemaphoreType.DMA,
        ],
    )(x, ei_flat, ea_t, wb)


def _xw_body(x_ref, wn_ref, bn_ref, o_ref):
    o_ref[...] = (jnp.dot(x_ref[...], wn_ref[...],
                          preferred_element_type=jnp.float32) + bn_ref[...])


def _xw(x, Wn, bn):
    BN = 2000
    return pl.pallas_call(
        _xw_body,
        grid=(N // BN,),
        in_specs=[
            pl.BlockSpec((BN, D), lambda i: (i, 0)),
            pl.BlockSpec((D, D), lambda i: (0, 0)),
            pl.BlockSpec((1, D), lambda i: (0, 0)),
        ],
        out_specs=pl.BlockSpec((BN, D), lambda i: (i, 0)),
        out_shape=jax.ShapeDtypeStruct((N, D), jnp.float32),
    )(x, Wn[:D], bn.reshape(1, D))


def _out_body(xw_ref, p0_ref, p1_ref, w2_ref, o_ref):
    agg = p0_ref[...] + p1_ref[...]
    o_ref[...] = xw_ref[...] + jnp.dot(agg, w2_ref[...],
                                       preferred_element_type=jnp.float32)


def _node_update(xw, p0, p1, Wn):
    BN = 2000
    return pl.pallas_call(
        _out_body,
        grid=(N // BN,),
        in_specs=[
            pl.BlockSpec((BN, D), lambda i: (i, 0)),
            pl.BlockSpec((BN, D), lambda i: (i, 0)),
            pl.BlockSpec((BN, D), lambda i: (i, 0)),
            pl.BlockSpec((D, D), lambda i: (0, 0)),
        ],
        out_specs=pl.BlockSpec((BN, D), lambda i: (i, 0)),
        out_shape=jax.ShapeDtypeStruct((N, D), jnp.float32),
    )(xw, p0, p1, Wn[D:])


def kernel(x, edge_index, edge_attr, we, beg, Wn, bn):
    ei_flat = edge_index.reshape(2 * E)
    ea_t = edge_attr.T
    wb = jnp.concatenate([we.reshape(DE), beg,
                          jnp.zeros((16 - DE - 1,), jnp.float32)])
    xw = _xw(x, Wn, bn)
    p0, p1 = _sc_aggregate(x, ei_flat, ea_t, wb)
    return _node_update(xw, p0, p1, Wn)
